# Initial kernel scaffold; baseline (speedup 1.0000x reference)
import functools

import jax
import jax.numpy as jnp
from jax import lax
from jax.experimental import pallas as pl
from jax.experimental.pallas import tpu as pltpu
from jax.experimental.pallas import tpu_sc as plsc

_NC = 2
_NS = 16
_LANES = 16
_NW = _NC * _NS
_CHUNK = 128
_NBUF = 2


def _ceil_to(x, m):
    return (x + m - 1) // m * m


def _row_block(n):
    for bn in (2000, 1000, 800, 500, 400, 250, 200, 128, 8):
        if n % bn == 0:
            return bn
    return n


def _relation_transform(node_states, W_rel):
    N, D = node_states.shape
    R = W_rel.shape[0]
    BN = _row_block(N)
    NB = N // BN

    def body(x_ref, w_ref, o_ref):
        o_ref[...] = lax.dot_general(
            x_ref[...], w_ref[0],
            dimension_numbers=(((1,), (1,)), ((), ())),
            preferred_element_type=jnp.float32)

    return pl.pallas_call(
        body,
        grid=(R, NB),
        in_specs=[
            pl.BlockSpec((BN, D), lambda r, i: (i, 0)),
            pl.BlockSpec((1, D, D), lambda r, i: (r, 0, 0)),
        ],
        out_specs=pl.BlockSpec((BN, D), lambda r, i: (r * NB + i, 0)),
        out_shape=jax.ShapeDtypeStruct((R * N, D), jnp.float32),
    )(node_states, W_rel)


def _self_transform(node_states, W_self, b_self2d):
    N, D = node_states.shape
    BN = _row_block(N)

    def body(x_ref, w_ref, b_ref, o_ref):
        o_ref[...] = lax.dot_general(
            x_ref[...], w_ref[...],
            dimension_numbers=(((1,), (1,)), ((), ())),
            preferred_element_type=jnp.float32) + b_ref[...]

    return pl.pallas_call(
        body,
        grid=(N // BN,),
        in_specs=[
            pl.BlockSpec((BN, D), lambda i: (i, 0)),
            pl.BlockSpec((D, D), lambda i: (0, 0)),
            pl.BlockSpec((1, D), lambda i: (0, 0)),
        ],
        out_specs=pl.BlockSpec((BN, D), lambda i: (i, 0)),
        out_shape=jax.ShapeDtypeStruct((N, D), jnp.float32),
    )(node_states, W_self, b_self2d)


def _edge_indices(src_r, et_r, n_nodes):
    def body(s_ref, t_ref, o_ref):
        o_ref[...] = t_ref[...] * n_nodes + s_ref[...]

    return pl.pallas_call(
        body,
        out_shape=jax.ShapeDtypeStruct(src_r.shape, jnp.int32),
    )(src_r, et_r)


def _sc_aggregate(H, packed, NP, D):
    CT = packed.shape[0]
    CPT = CT // _NW
    ROWS = NP // _NS
    mesh = plsc.VectorSubcoreMesh(core_axis_name="c", subcore_axis_name="s")

    @functools.partial(
        pl.kernel,
        out_type=[
            jax.ShapeDtypeStruct((_NC, NP, D), jnp.float32),
            jax.ShapeDtypeStruct((_NC, NP, _LANES), jnp.float32),
        ],
        mesh=mesh,
        scratch_types=[
            pltpu.VMEM((CPT, 2, _CHUNK), jnp.int32),
            pltpu.VMEM((_CHUNK, D), jnp.float32),
            pltpu.VMEM((_CHUNK, D), jnp.float32),
            pltpu.VMEM((_CHUNK, _LANES), jnp.float32),
            pltpu.VMEM((_CHUNK, _LANES), jnp.float32),
            pltpu.VMEM_SHARED((NP, D), jnp.float32),
            pltpu.VMEM_SHARED((NP, _LANES), jnp.float32),
            pltpu.SemaphoreType.DMA,
            pltpu.SemaphoreType.DMA,
            pltpu.SemaphoreType.DMA,
        ],
    )
    def k(h_hbm, packed_hbm, agg_hbm, deg_hbm,
          idx_all, rows0, rows1, ones_v, zdeg_v, agg_sh, deg_sh,
          sem, gsem0, gsem1):
        c = lax.axis_index("c")
        s = lax.axis_index("s")
        w = s * _NC + c

        zeros16 = jnp.zeros((_LANES,), jnp.float32)
        ones16 = jnp.ones((_LANES,), jnp.float32)

        @pl.loop(0, _CHUNK)
        def _(i):
            ones_v[i, :] = ones16
            zdeg_v[i, :] = zeros16

            @pl.loop(0, D, step=_LANES)
            def _(j):
                rows0[i, pl.ds(j, _LANES)] = zeros16

        base = s * ROWS
        for kk in range(ROWS // _CHUNK):
            pltpu.sync_copy(rows0, agg_sh.at[pl.ds(base + kk * _CHUNK, _CHUNK)])
            pltpu.sync_copy(zdeg_v, deg_sh.at[pl.ds(base + kk * _CHUNK, _CHUNK)])
        plsc.subcore_barrier()

        pltpu.async_copy(packed_hbm.at[pl.ds(w * CPT, CPT)], idx_all, sem).wait()

        rows = (rows0, rows1)
        gsems = (gsem0, gsem1)
        for b in range(_NBUF):
            pltpu.async_copy(h_hbm.at[idx_all.at[b, 0]], rows[b], gsems[b])

        @pl.loop(0, CPT, step=_NBUF)
        def _(k0):
            for b in range(_NBUF):
                kb = k0 + b
                pltpu.make_async_copy(
                    h_hbm.at[idx_all.at[kb, 0]], rows[b], gsems[b]).wait()
                pltpu.sync_copy(rows[b], agg_sh.at[idx_all.at[kb, 1]], add=True)
                pltpu.sync_copy(ones_v, deg_sh.at[idx_all.at[kb, 1]], add=True)

                @pl.when(kb + _NBUF < CPT)
                def _():
                    pltpu.async_copy(
                        h_hbm.at[idx_all.at[kb + _NBUF, 0]], rows[b], gsems[b])

        plsc.subcore_barrier()

        pltpu.sync_copy(agg_sh.at[pl.ds(base, ROWS)],
                        agg_hbm.at[c, pl.ds(base, ROWS)])
        pltpu.sync_copy(deg_sh.at[pl.ds(base, ROWS)],
                        deg_hbm.at[c, pl.ds(base, ROWS)])

    return k(H, packed)


def _finalize(selfed, agg_p, deg_p):
    N, D = selfed.shape
    BN = _row_block(N)

    def body(s_ref, a_ref, d_ref, o_ref):
        a = a_ref[0] + a_ref[1]
        dg = d_ref[0, :, 0:1] + d_ref[1, :, 0:1]
        o_ref[...] = jnp.maximum(s_ref[...] + a / jnp.maximum(dg, 1.0), 0.0)

    return pl.pallas_call(
        body,
        grid=(N // BN,),
        in_specs=[
            pl.BlockSpec((BN, D), lambda i: (i, 0)),
            pl.BlockSpec((2, BN, D), lambda i: (0, i, 0)),
            pl.BlockSpec((2, BN, _LANES), lambda i: (0, i, 0)),
        ],
        out_specs=pl.BlockSpec((BN, D), lambda i: (i, 0)),
        out_shape=jax.ShapeDtypeStruct((N, D), jnp.float32),
    )(selfed, agg_p, deg_p)


def kernel(node_states, edge_index, edge_type, W_self, b_self, W_rel):
    N, D = node_states.shape
    E = edge_type.shape[0]

    NP = _ceil_to(N, _NS * _CHUNK)
    Ep = _ceil_to(E, _NW * _CHUNK * _NBUF)
    CT = Ep // _CHUNK

    src = edge_index[0]
    dst = edge_index[1]
    pad = Ep - E
    if pad:
        src = jnp.concatenate([src, jnp.zeros((pad,), jnp.int32)])
        edge_type = jnp.concatenate([edge_type, jnp.zeros((pad,), jnp.int32)])
        dst = jnp.concatenate([dst, jnp.full((pad,), NP - 1, jnp.int32)])

    src_r = src.reshape(CT, _CHUNK)
    et_r = edge_type.reshape(CT, _CHUNK)
    dst_r = dst.reshape(CT, _CHUNK)

    gidx = _edge_indices(src_r, et_r, N)
    packed = jnp.stack([gidx, dst_r], axis=1)

    H = _relation_transform(node_states, W_rel)
    agg_p, deg_p = _sc_aggregate(H, packed, NP, D)
    selfed = _self_transform(node_states, W_self, b_self.reshape(1, D))
    return _finalize(selfed, agg_p[:, :N], deg_p[:, :N])


# speedup vs baseline: 12.0645x; 12.0645x over previous
import dataclasses
import functools

import jax
import jax.numpy as jnp
from jax import lax
from jax.experimental import pallas as pl
from jax.experimental.pallas import tpu as pltpu
from jax.experimental.pallas import tpu_sc as plsc

_NC = 2
_NS = 16
_LANES = 16
_NW = _NC * _NS
_CHUNK = 128
_NBUF = 2


def _ceil_to(x, m):
    return (x + m - 1) // m * m


def _row_block(n):
    for bn in (2000, 1000, 800, 500, 400, 250, 200, 128, 8):
        if n % bn == 0:
            return bn
    return n


def _relation_transform(node_states, W_rel):
    N, D = node_states.shape
    R = W_rel.shape[0]
    BN = _row_block(N)
    NB = N // BN

    def body(x_ref, w_ref, o_ref):
        o_ref[...] = lax.dot_general(
            x_ref[...], w_ref[0],
            dimension_numbers=(((1,), (1,)), ((), ())),
            preferred_element_type=jnp.float32)

    return pl.pallas_call(
        body,
        grid=(R, NB),
        in_specs=[
            pl.BlockSpec((BN, D), lambda r, i: (i, 0)),
            pl.BlockSpec((1, D, D), lambda r, i: (r, 0, 0)),
        ],
        out_specs=pl.BlockSpec((BN, D), lambda r, i: (r * NB + i, 0)),
        out_shape=jax.ShapeDtypeStruct((R * N, D), jnp.float32),
    )(node_states, W_rel)


def _self_transform(node_states, W_self, b_self2d):
    N, D = node_states.shape
    BN = _row_block(N)

    def body(x_ref, w_ref, b_ref, o_ref):
        o_ref[...] = lax.dot_general(
            x_ref[...], w_ref[...],
            dimension_numbers=(((1,), (1,)), ((), ())),
            preferred_element_type=jnp.float32) + b_ref[...]

    return pl.pallas_call(
        body,
        grid=(N // BN,),
        in_specs=[
            pl.BlockSpec((BN, D), lambda i: (i, 0)),
            pl.BlockSpec((D, D), lambda i: (0, 0)),
            pl.BlockSpec((1, D), lambda i: (0, 0)),
        ],
        out_specs=pl.BlockSpec((BN, D), lambda i: (i, 0)),
        out_shape=jax.ShapeDtypeStruct((N, D), jnp.float32),
    )(node_states, W_self, b_self2d)


def _edge_indices(src_r, et_r, n_nodes):
    def body(s_ref, t_ref, o_ref):
        o_ref[...] = t_ref[...] * n_nodes + s_ref[...]

    return pl.pallas_call(
        body,
        out_shape=jax.ShapeDtypeStruct(src_r.shape, jnp.int32),
    )(src_r, et_r)


def _sc_aggregate(H, packed, NP, D):
    CT = packed.shape[0]
    CPT = CT // _NW
    ROWS = NP // _NS
    mesh = plsc.VectorSubcoreMesh(core_axis_name="c", subcore_axis_name="s")
    cp = pltpu.CompilerParams()
    if "needs_layout_passes" in pltpu.CompilerParams.__dataclass_fields__:
        cp = dataclasses.replace(cp, needs_layout_passes=False)

    @functools.partial(
        pl.kernel,
        compiler_params=cp,
        out_type=[
            jax.ShapeDtypeStruct((_NC, NP, D), jnp.float32),
            jax.ShapeDtypeStruct((_NW, NP), jnp.float32),
        ],
        mesh=mesh,
        scratch_types=[
            pltpu.VMEM((2, _CHUNK), jnp.int32),
            pltpu.VMEM((2, _CHUNK), jnp.int32),
            pltpu.VMEM((_CHUNK, D), jnp.float32),
            pltpu.VMEM((_CHUNK, D), jnp.float32),
            pltpu.VMEM((NP,), jnp.float32),
            pltpu.VMEM_SHARED((NP, D), jnp.float32),
            pltpu.SemaphoreType.DMA,
            pltpu.SemaphoreType.DMA,
            pltpu.SemaphoreType.DMA,
            pltpu.SemaphoreType.DMA,
        ],
    )
    def k(h_hbm, packed_hbm, agg_hbm, deg_hbm,
          idx0, idx1, rows0, rows1, deg_local, agg_sh,
          isem0, isem1, gsem0, gsem1):
        c = lax.axis_index("c")
        s = lax.axis_index("s")
        w = s * _NC + c
        cbase = w * CPT

        zeros16 = jnp.zeros((_LANES,), jnp.float32)
        ones16 = jnp.ones((_LANES,), jnp.float32)

        @pl.loop(0, NP, step=_LANES)
        def _(i):
            deg_local[pl.ds(i, _LANES)] = zeros16

        @pl.loop(0, _CHUNK)
        def _(i):
            @pl.loop(0, D, step=_LANES)
            def _(j):
                rows0[i, pl.ds(j, _LANES)] = zeros16

        base = s * ROWS
        for kk in range(ROWS // _CHUNK):
            pltpu.sync_copy(rows0, agg_sh.at[pl.ds(base + kk * _CHUNK, _CHUNK)])
        plsc.subcore_barrier()

        idxs = (idx0, idx1)
        rows = (rows0, rows1)
        isems = (isem0, isem1)
        gsems = (gsem0, gsem1)

        pltpu.sync_copy(packed_hbm.at[cbase], idx0)
        pltpu.async_copy(packed_hbm.at[cbase + 1], idx1, isem1)
        pltpu.async_copy(h_hbm.at[idx0.at[0]], rows0, gsem0)

        @pl.loop(0, CPT, step=_NBUF)
        def _(k0):
            for b in range(_NBUF):
                kb = k0 + b
                pltpu.make_async_copy(
                    h_hbm.at[idxs[b].at[0]], rows[b], gsems[b]).wait()
                pltpu.sync_copy(rows[b], agg_sh.at[idxs[b].at[1]], add=True)
                for jj in range(_CHUNK // _LANES):
                    idx16 = idxs[b][1, pl.ds(jj * _LANES, _LANES)]
                    plsc.addupdate_scatter(deg_local, [idx16], ones16)

                @pl.when(kb + 2 < CPT)
                def _():
                    pltpu.async_copy(
                        packed_hbm.at[cbase + kb + 2], idxs[b], isems[b])

                @pl.when(kb + 1 < CPT)
                def _():
                    pltpu.make_async_copy(
                        packed_hbm.at[cbase + kb + 1],
                        idxs[b ^ 1], isems[b ^ 1]).wait()
                    pltpu.async_copy(
                        h_hbm.at[idxs[b ^ 1].at[0]], rows[b ^ 1], gsems[b ^ 1])

        plsc.subcore_barrier()

        pltpu.sync_copy(agg_sh.at[pl.ds(base, ROWS)],
                        agg_hbm.at[c, pl.ds(base, ROWS)])
        pltpu.sync_copy(deg_local, deg_hbm.at[w])

    return k(H, packed)


def _deg_sum(deg_p, NP):
    NPB = 2048

    def body(d_ref, o_ref):
        o_ref[...] = jnp.sum(d_ref[...], axis=0)[:, None]

    return pl.pallas_call(
        body,
        grid=(NP // NPB,),
        in_specs=[pl.BlockSpec((_NW, NPB), lambda i: (0, i))],
        out_specs=pl.BlockSpec((NPB, 1), lambda i: (i, 0)),
        out_shape=jax.ShapeDtypeStruct((NP, 1), jnp.float32),
    )(deg_p)


def _finalize(selfed, agg_p, deg_n1):
    N, D = selfed.shape
    BN = _row_block(N)

    def body(s_ref, a_ref, d_ref, o_ref):
        a = a_ref[0] + a_ref[1]
        o_ref[...] = jnp.maximum(
            s_ref[...] + a / jnp.maximum(d_ref[...], 1.0), 0.0)

    return pl.pallas_call(
        body,
        grid=(N // BN,),
        in_specs=[
            pl.BlockSpec((BN, D), lambda i: (i, 0)),
            pl.BlockSpec((2, BN, D), lambda i: (0, i, 0)),
            pl.BlockSpec((BN, 1), lambda i: (i, 0)),
        ],
        out_specs=pl.BlockSpec((BN, D), lambda i: (i, 0)),
        out_shape=jax.ShapeDtypeStruct((N, D), jnp.float32),
    )(selfed, agg_p, deg_n1)


def kernel(node_states, edge_index, edge_type, W_self, b_self, W_rel):
    N, D = node_states.shape
    E = edge_type.shape[0]

    NP = _ceil_to(N, _NS * _CHUNK)
    Ep = _ceil_to(E, _NW * _CHUNK * _NBUF)
    CT = Ep // _CHUNK

    src = edge_index[0]
    dst = edge_index[1]
    pad = Ep - E
    if pad:
        src = jnp.concatenate([src, jnp.zeros((pad,), jnp.int32)])
        edge_type = jnp.concatenate([edge_type, jnp.zeros((pad,), jnp.int32)])
        dst = jnp.concatenate([dst, jnp.full((pad,), NP - 1, jnp.int32)])

    src_r = src.reshape(CT, _CHUNK)
    et_r = edge_type.reshape(CT, _CHUNK)
    dst_r = dst.reshape(CT, _CHUNK)

    gidx = _edge_indices(src_r, et_r, N)
    packed = jnp.stack([gidx, dst_r], axis=1)

    H = _relation_transform(node_states, W_rel)
    agg_p, deg_p = _sc_aggregate(H, packed, NP, D)
    deg = _deg_sum(deg_p, NP)
    selfed = _self_transform(node_states, W_self, b_self.reshape(1, D))
    return _finalize(selfed, agg_p[:, :N], deg[:N])


# speedup vs baseline: 13.4016x; 1.1108x over previous
import dataclasses
import functools

import jax
import jax.numpy as jnp
from jax import lax
from jax.experimental import pallas as pl
from jax.experimental.pallas import tpu as pltpu
from jax.experimental.pallas import tpu_sc as plsc

_NC = 2
_NS = 16
_LANES = 16
_NW = _NC * _NS
_CHUNK = 128
_NBUF = 2


def _ceil_to(x, m):
    return (x + m - 1) // m * m


def _row_block(n):
    for bn in (2000, 1000, 800, 500, 400, 250, 200, 128, 8):
        if n % bn == 0:
            return bn
    return n


def _relation_transform(node_states, W_rel):
    N, D = node_states.shape
    R = W_rel.shape[0]
    BN = _row_block(N)
    NB = N // BN

    def body(x_ref, w_ref, o_ref):
        o_ref[...] = lax.dot_general(
            x_ref[...], w_ref[0],
            dimension_numbers=(((1,), (1,)), ((), ())),
            preferred_element_type=jnp.float32)

    return pl.pallas_call(
        body,
        grid=(R, NB),
        in_specs=[
            pl.BlockSpec((BN, D), lambda r, i: (i, 0)),
            pl.BlockSpec((1, D, D), lambda r, i: (r, 0, 0)),
        ],
        out_specs=pl.BlockSpec((BN, D), lambda r, i: (r * NB + i, 0)),
        out_shape=jax.ShapeDtypeStruct((R * N, D), jnp.float32),
    )(node_states, W_rel)


def _self_transform(node_states, W_self, b_self2d):
    N, D = node_states.shape
    BN = _row_block(N)

    def body(x_ref, w_ref, b_ref, o_ref):
        o_ref[...] = lax.dot_general(
            x_ref[...], w_ref[...],
            dimension_numbers=(((1,), (1,)), ((), ())),
            preferred_element_type=jnp.float32) + b_ref[...]

    return pl.pallas_call(
        body,
        grid=(N // BN,),
        in_specs=[
            pl.BlockSpec((BN, D), lambda i: (i, 0)),
            pl.BlockSpec((D, D), lambda i: (0, 0)),
            pl.BlockSpec((1, D), lambda i: (0, 0)),
        ],
        out_specs=pl.BlockSpec((BN, D), lambda i: (i, 0)),
        out_shape=jax.ShapeDtypeStruct((N, D), jnp.float32),
    )(node_states, W_self, b_self2d)


def _edge_indices(src_r, et_r, n_nodes):
    def body(s_ref, t_ref, o_ref):
        o_ref[...] = t_ref[...] * n_nodes + s_ref[...]

    return pl.pallas_call(
        body,
        out_shape=jax.ShapeDtypeStruct(src_r.shape, jnp.int32),
    )(src_r, et_r)


def _sc_aggregate(H, packed, NP, D):
    CT = packed.shape[0]
    CPT = CT // _NW
    ROWS = NP // _NS
    mesh = plsc.VectorSubcoreMesh(core_axis_name="c", subcore_axis_name="s")
    cp = pltpu.CompilerParams()
    if "needs_layout_passes" in pltpu.CompilerParams.__dataclass_fields__:
        cp = dataclasses.replace(cp, needs_layout_passes=False)

    @functools.partial(
        pl.kernel,
        compiler_params=cp,
        out_type=[
            jax.ShapeDtypeStruct((_NC, NP, D), jnp.float32),
            jax.ShapeDtypeStruct((_NW, NP), jnp.float32),
        ],
        mesh=mesh,
        scratch_types=[
            pltpu.VMEM((2, _CHUNK), jnp.int32),
            pltpu.VMEM((2, _CHUNK), jnp.int32),
            pltpu.VMEM((_CHUNK, D), jnp.float32),
            pltpu.VMEM((_CHUNK, D), jnp.float32),
            pltpu.VMEM((NP,), jnp.float32),
            pltpu.VMEM_SHARED((NP, D), jnp.float32),
            pltpu.SemaphoreType.DMA,
            pltpu.SemaphoreType.DMA,
            pltpu.SemaphoreType.DMA,
            pltpu.SemaphoreType.DMA,
        ],
    )
    def k(h_hbm, packed_hbm, agg_hbm, deg_hbm,
          idx0, idx1, rows0, rows1, deg_local, agg_sh,
          isem0, isem1, gsem0, gsem1):
        c = lax.axis_index("c")
        s = lax.axis_index("s")
        w = s * _NC + c
        cbase = w * CPT

        zeros16 = jnp.zeros((_LANES,), jnp.float32)
        ones16 = jnp.ones((_LANES,), jnp.float32)

        @pl.loop(0, NP, step=_LANES)
        def _(i):
            deg_local[pl.ds(i, _LANES)] = zeros16

        @pl.loop(0, _CHUNK)
        def _(i):
            @pl.loop(0, D, step=_LANES)
            def _(j):
                rows0[i, pl.ds(j, _LANES)] = zeros16

        base = s * ROWS
        for kk in range(ROWS // _CHUNK):
            pltpu.sync_copy(rows0, agg_sh.at[pl.ds(base + kk * _CHUNK, _CHUNK)])
        plsc.subcore_barrier()

        idxs = (idx0, idx1)
        rows = (rows0, rows1)
        isems = (isem0, isem1)
        gsems = (gsem0, gsem1)

        pltpu.sync_copy(packed_hbm.at[cbase], idx0)
        pltpu.async_copy(packed_hbm.at[cbase + 1], idx1, isem1)
        pltpu.async_copy(h_hbm.at[idx0.at[0]], rows0, gsem0)

        @pl.loop(0, CPT, step=_NBUF)
        def _(k0):
            for b in range(_NBUF):
                kb = k0 + b
                pltpu.make_async_copy(
                    h_hbm.at[idxs[b].at[0]], rows[b], gsems[b]).wait()

                @pl.when(kb + 1 < CPT)
                def _():
                    pltpu.make_async_copy(
                        packed_hbm.at[cbase + kb + 1],
                        idxs[b ^ 1], isems[b ^ 1]).wait()
                    pltpu.async_copy(
                        h_hbm.at[idxs[b ^ 1].at[0]], rows[b ^ 1], gsems[b ^ 1])

                pltpu.sync_copy(rows[b], agg_sh.at[idxs[b].at[1]], add=True)
                for jj in range(_CHUNK // _LANES):
                    idx16 = idxs[b][1, pl.ds(jj * _LANES, _LANES)]
                    plsc.addupdate_scatter(deg_local, [idx16], ones16)

                @pl.when(kb + 2 < CPT)
                def _():
                    pltpu.async_copy(
                        packed_hbm.at[cbase + kb + 2], idxs[b], isems[b])

        plsc.subcore_barrier()

        pltpu.sync_copy(agg_sh.at[pl.ds(base, ROWS)],
                        agg_hbm.at[c, pl.ds(base, ROWS)])
        pltpu.sync_copy(deg_local, deg_hbm.at[w])

    return k(H, packed)


def _deg_sum(deg_p, NP):
    NPB = 2048

    def body(d_ref, o_ref):
        o_ref[...] = jnp.sum(d_ref[...], axis=0)[:, None]

    return pl.pallas_call(
        body,
        grid=(NP // NPB,),
        in_specs=[pl.BlockSpec((_NW, NPB), lambda i: (0, i))],
        out_specs=pl.BlockSpec((NPB, 1), lambda i: (i, 0)),
        out_shape=jax.ShapeDtypeStruct((NP, 1), jnp.float32),
    )(deg_p)


def _finalize(selfed, agg_p, deg_n1):
    N, D = selfed.shape
    BN = _row_block(N)

    def body(s_ref, a_ref, d_ref, o_ref):
        a = a_ref[0] + a_ref[1]
        o_ref[...] = jnp.maximum(
            s_ref[...] + a / jnp.maximum(d_ref[...], 1.0), 0.0)

    return pl.pallas_call(
        body,
        grid=(N // BN,),
        in_specs=[
            pl.BlockSpec((BN, D), lambda i: (i, 0)),
            pl.BlockSpec((2, BN, D), lambda i: (0, i, 0)),
            pl.BlockSpec((BN, 1), lambda i: (i, 0)),
        ],
        out_specs=pl.BlockSpec((BN, D), lambda i: (i, 0)),
        out_shape=jax.ShapeDtypeStruct((N, D), jnp.float32),
    )(selfed, agg_p, deg_n1)


def kernel(node_states, edge_index, edge_type, W_self, b_self, W_rel):
    N, D = node_states.shape
    E = edge_type.shape[0]

    NP = _ceil_to(N, _NS * _CHUNK)
    Ep = _ceil_to(E, _NW * _CHUNK * _NBUF)
    CT = Ep // _CHUNK

    src = edge_index[0]
    dst = edge_index[1]
    pad = Ep - E
    if pad:
        src = jnp.concatenate([src, jnp.zeros((pad,), jnp.int32)])
        edge_type = jnp.concatenate([edge_type, jnp.zeros((pad,), jnp.int32)])
        dst = jnp.concatenate([dst, jnp.full((pad,), NP - 1, jnp.int32)])

    src_r = src.reshape(CT, _CHUNK)
    et_r = edge_type.reshape(CT, _CHUNK)
    dst_r = dst.reshape(CT, _CHUNK)

    gidx = _edge_indices(src_r, et_r, N)
    packed = jnp.stack([gidx, dst_r], axis=1)

    H = _relation_transform(node_states, W_rel)
    agg_p, deg_p = _sc_aggregate(H, packed, NP, D)
    deg = _deg_sum(deg_p, NP)
    selfed = _self_transform(node_states, W_self, b_self.reshape(1, D))
    return _finalize(selfed, agg_p[:, :N], deg[:N])


# speedup vs baseline: 13.9168x; 1.0384x over previous
import dataclasses
import functools

import jax
import jax.numpy as jnp
from jax import lax
from jax.experimental import pallas as pl
from jax.experimental.pallas import tpu as pltpu
from jax.experimental.pallas import tpu_sc as plsc

_NC = 2
_NS = 16
_LANES = 16
_NW = _NC * _NS
_CHUNK = 128
_NBUF = 2


def _ceil_to(x, m):
    return (x + m - 1) // m * m


def _row_block(n):
    for bn in (2000, 1000, 800, 500, 400, 250, 200, 128, 8):
        if n % bn == 0:
            return bn
    return n


def _relation_transform(node_states, W_rel):
    N, D = node_states.shape
    R = W_rel.shape[0]
    BN = _row_block(N)
    NB = N // BN

    def body(x_ref, w_ref, o_ref):
        o_ref[...] = lax.dot_general(
            x_ref[...], w_ref[0],
            dimension_numbers=(((1,), (1,)), ((), ())),
            preferred_element_type=jnp.float32)

    return pl.pallas_call(
        body,
        grid=(R, NB),
        in_specs=[
            pl.BlockSpec((BN, D), lambda r, i: (i, 0)),
            pl.BlockSpec((1, D, D), lambda r, i: (r, 0, 0)),
        ],
        out_specs=pl.BlockSpec((BN, D), lambda r, i: (r * NB + i, 0)),
        out_shape=jax.ShapeDtypeStruct((R * N, D), jnp.float32),
    )(node_states, W_rel)


def _self_transform(node_states, W_self, b_self2d):
    N, D = node_states.shape
    BN = _row_block(N)

    def body(x_ref, w_ref, b_ref, o_ref):
        o_ref[...] = lax.dot_general(
            x_ref[...], w_ref[...],
            dimension_numbers=(((1,), (1,)), ((), ())),
            preferred_element_type=jnp.float32) + b_ref[...]

    return pl.pallas_call(
        body,
        grid=(N // BN,),
        in_specs=[
            pl.BlockSpec((BN, D), lambda i: (i, 0)),
            pl.BlockSpec((D, D), lambda i: (0, 0)),
            pl.BlockSpec((1, D), lambda i: (0, 0)),
        ],
        out_specs=pl.BlockSpec((BN, D), lambda i: (i, 0)),
        out_shape=jax.ShapeDtypeStruct((N, D), jnp.float32),
    )(node_states, W_self, b_self2d)


def _edge_indices(src_r, et_r, n_nodes):
    def body(s_ref, t_ref, o_ref):
        o_ref[...] = t_ref[...] * n_nodes + s_ref[...]

    return pl.pallas_call(
        body,
        out_shape=jax.ShapeDtypeStruct(src_r.shape, jnp.int32),
    )(src_r, et_r)


def _sc_aggregate(H, packed, NP, D):
    CT = packed.shape[0]
    CPP = CT // _NW * _NC
    CPT0 = int(round(CPP * 0.725 / _NBUF)) * _NBUF
    CPT1 = CPP - CPT0
    NCH0 = CPT0 * _NS
    ROWS = NP // _NS
    mesh = plsc.VectorSubcoreMesh(core_axis_name="c", subcore_axis_name="s")
    cp = pltpu.CompilerParams()
    if "needs_layout_passes" in pltpu.CompilerParams.__dataclass_fields__:
        cp = dataclasses.replace(cp, needs_layout_passes=False)

    @functools.partial(
        pl.kernel,
        compiler_params=cp,
        out_type=[
            jax.ShapeDtypeStruct((_NC, NP, D), jnp.float32),
            jax.ShapeDtypeStruct((_NW, NP), jnp.float32),
        ],
        mesh=mesh,
        scratch_types=[
            pltpu.VMEM((2, _CHUNK), jnp.int32),
            pltpu.VMEM((2, _CHUNK), jnp.int32),
            pltpu.VMEM((_CHUNK, D), jnp.float32),
            pltpu.VMEM((_CHUNK, D), jnp.float32),
            pltpu.VMEM((NP,), jnp.float32),
            pltpu.VMEM_SHARED((NP, D), jnp.float32),
            pltpu.SemaphoreType.DMA,
            pltpu.SemaphoreType.DMA,
            pltpu.SemaphoreType.DMA,
            pltpu.SemaphoreType.DMA,
        ],
    )
    def k(h_hbm, packed_hbm, agg_hbm, deg_hbm,
          idx0, idx1, rows0, rows1, deg_local, agg_sh,
          isem0, isem1, gsem0, gsem1):
        c = lax.axis_index("c")
        s = lax.axis_index("s")

        zeros16 = jnp.zeros((_LANES,), jnp.float32)
        ones16 = jnp.ones((_LANES,), jnp.float32)

        @pl.loop(0, NP, step=_LANES)
        def _(i):
            deg_local[pl.ds(i, _LANES)] = zeros16

        @pl.loop(0, _CHUNK)
        def _(i):
            @pl.loop(0, D, step=_LANES)
            def _(j):
                rows0[i, pl.ds(j, _LANES)] = zeros16

        base = s * ROWS
        for kk in range(ROWS // _CHUNK):
            pltpu.sync_copy(rows0, agg_sh.at[pl.ds(base + kk * _CHUNK, _CHUNK)])
        plsc.subcore_barrier()

        idxs = (idx0, idx1)
        rows = (rows0, rows1)
        isems = (isem0, isem1)
        gsems = (gsem0, gsem1)

        def pipeline(cbase, cpt):
            pltpu.sync_copy(packed_hbm.at[cbase], idx0)
            pltpu.async_copy(packed_hbm.at[cbase + 1], idx1, isem1)
            pltpu.async_copy(h_hbm.at[idx0.at[0]], rows0, gsem0)

            @pl.loop(0, cpt, step=_NBUF)
            def _(k0):
                for b in range(_NBUF):
                    kb = k0 + b
                    pltpu.make_async_copy(
                        h_hbm.at[idxs[b].at[0]], rows[b], gsems[b]).wait()

                    @pl.when(kb + 1 < cpt)
                    def _():
                        pltpu.make_async_copy(
                            packed_hbm.at[cbase + kb + 1],
                            idxs[b ^ 1], isems[b ^ 1]).wait()
                        pltpu.async_copy(
                            h_hbm.at[idxs[b ^ 1].at[0]],
                            rows[b ^ 1], gsems[b ^ 1])

                    pltpu.sync_copy(
                        rows[b], agg_sh.at[idxs[b].at[1]], add=True)
                    for jj in range(_CHUNK // _LANES):
                        idx16 = idxs[b][1, pl.ds(jj * _LANES, _LANES)]
                        plsc.addupdate_scatter(deg_local, [idx16], ones16)

                    @pl.when(kb + 2 < cpt)
                    def _():
                        pltpu.async_copy(
                            packed_hbm.at[cbase + kb + 2], idxs[b], isems[b])

        @pl.when(c == 0)
        def _():
            pipeline(s * CPT0, CPT0)

        @pl.when(c == 1)
        def _():
            pipeline(NCH0 + s * CPT1, CPT1)

        plsc.subcore_barrier()

        pltpu.sync_copy(agg_sh.at[pl.ds(base, ROWS)],
                        agg_hbm.at[c, pl.ds(base, ROWS)])
        pltpu.sync_copy(deg_local, deg_hbm.at[s * _NC + c])

    return k(H, packed)


def _deg_sum(deg_p, NP):
    NPB = 2048

    def body(d_ref, o_ref):
        o_ref[...] = jnp.sum(d_ref[...], axis=0)[:, None]

    return pl.pallas_call(
        body,
        grid=(NP // NPB,),
        in_specs=[pl.BlockSpec((_NW, NPB), lambda i: (0, i))],
        out_specs=pl.BlockSpec((NPB, 1), lambda i: (i, 0)),
        out_shape=jax.ShapeDtypeStruct((NP, 1), jnp.float32),
    )(deg_p)


def _finalize(selfed, agg_p, deg_n1):
    N, D = selfed.shape
    BN = _row_block(N)

    def body(s_ref, a_ref, d_ref, o_ref):
        a = a_ref[0] + a_ref[1]
        o_ref[...] = jnp.maximum(
            s_ref[...] + a / jnp.maximum(d_ref[...], 1.0), 0.0)

    return pl.pallas_call(
        body,
        grid=(N // BN,),
        in_specs=[
            pl.BlockSpec((BN, D), lambda i: (i, 0)),
            pl.BlockSpec((2, BN, D), lambda i: (0, i, 0)),
            pl.BlockSpec((BN, 1), lambda i: (i, 0)),
        ],
        out_specs=pl.BlockSpec((BN, D), lambda i: (i, 0)),
        out_shape=jax.ShapeDtypeStruct((N, D), jnp.float32),
    )(selfed, agg_p, deg_n1)


def kernel(node_states, edge_index, edge_type, W_self, b_self, W_rel):
    N, D = node_states.shape
    E = edge_type.shape[0]

    NP = _ceil_to(N, _NS * _CHUNK)
    Ep = _ceil_to(E, _NW * _CHUNK * _NBUF)
    CT = Ep // _CHUNK

    src = edge_index[0]
    dst = edge_index[1]
    pad = Ep - E
    if pad:
        src = jnp.concatenate([src, jnp.zeros((pad,), jnp.int32)])
        edge_type = jnp.concatenate([edge_type, jnp.zeros((pad,), jnp.int32)])
        dst = jnp.concatenate([dst, jnp.full((pad,), NP - 1, jnp.int32)])

    src_r = src.reshape(CT, _CHUNK)
    et_r = edge_type.reshape(CT, _CHUNK)
    dst_r = dst.reshape(CT, _CHUNK)

    gidx = _edge_indices(src_r, et_r, N)
    packed = jnp.stack([gidx, dst_r], axis=1)

    H = _relation_transform(node_states, W_rel)
    agg_p, deg_p = _sc_aggregate(H, packed, NP, D)
    deg = _deg_sum(deg_p, NP)
    selfed = _self_transform(node_states, W_self, b_self.reshape(1, D))
    return _finalize(selfed, agg_p[:, :N], deg[:N])
